# baseline (speedup 1.0000x reference)
import functools

import jax
import jax.numpy as jnp
from jax import lax
from jax.experimental import pallas as pl
from jax.experimental.pallas import tpu as pltpu
from jax.experimental.pallas import tpu_sc as plsc

_BT = 512
_K = 4
_L = 16
_DM = 128
_DA = 64
_KL = 32
_PW = 128


def _geom_body(x_ref, xf_ref, Wl_ref, sig_ref, mu_ref, blift_ref,
               fb_ref, Wu_ref, bu_ref, Wk_ref, bk_ref, Wq_ref, bq_ref,
               pos_ref, Wz_ref, Wih_ref, bih_ref, Whh_ref, bhh_ref,
               gidx_ref, F_ref, P_ref, ay_ref, mem_ref, wl_s, bl_s):
    b = pl.program_id(0)
    i = pl.program_id(1)
    nblk = pl.num_programs(1)
    xr = x_ref[0]
    xf = xf_ref[0]
    T = xf.shape[0]
    D = xf.shape[1]

    @pl.when((b == 0) & (i == 0))
    def _fold():
        wl_s[...] = Wl_ref[...] / sig_ref[...]
        bl_s[...] = blift_ref[...] - lax.dot_general(
            mu_ref[...] / sig_ref[...], Wl_ref[...], (((0,), (0,)), ((), ())),
            preferred_element_type=jnp.float32)

    Wl = wl_s[...]
    b_l = bl_s[...]
    sq_r = jnp.sum(xr * xr, axis=1, keepdims=True)
    sq_c = jnp.sum(xf * xf, axis=1, keepdims=True).reshape(1, T)
    G2 = lax.dot_general(-2.0 * xr, xf, (((1,), (1,)), ((), ())),
                         preferred_element_type=jnp.float32)
    col = lax.broadcasted_iota(jnp.int32, (_BT, T), 1)
    rowg = i * _BT + lax.broadcasted_iota(jnp.int32, (_BT, T), 0)
    vals = jnp.where(col == rowg, 1e9, G2 + sq_c)
    dist_acc = jnp.zeros((_BT, 1), jnp.float32)
    for k in range(_K):
        m = jnp.min(vals, axis=1, keepdims=True)
        sel = vals == m
        idxk = jnp.min(jnp.where(sel, col, T), axis=1, keepdims=True)
        dist_acc = dist_acc + jnp.sqrt(jnp.maximum(m + sq_r, 0.0))
        gidx_ref[0, :, pl.ds(k, 1)] = idxk + b * T
        vals = jnp.where(sel, 3e9, vals)
    P_ref[0, :, :_KL] = jnp.dot(xr, Wl[D:2 * D],
                                preferred_element_type=jnp.float32)
    F_ref[0] = (jnp.dot(xr, Wl[:D], preferred_element_type=jnp.float32)
                + jnp.sqrt(sq_r) * Wl[2 * D:2 * D + 1]
                + (dist_acc * (1.0 / _K)) * Wl[2 * D + 1:2 * D + 2]
                + b_l)

    @pl.when(i == nblk - 1)
    def _router():
        u = jnp.tanh(jnp.dot(xf, Wu_ref[...],
                             preferred_element_type=jnp.float32) + bu_ref[...])
        keys = jnp.dot(u, Wk_ref[...],
                       preferred_element_type=jnp.float32) + bk_ref[...]
        keysT = keys.T
        fb = fb_ref[0]
        Wihm = Wih_ref[:_DM]
        gi_fb = fb * Wih_ref[_DM:_DM + 1] + bih_ref[...]
        pos = pos_ref[...]
        inv_sqrt_da = 1.0 / (_DA ** 0.5)
        mem_ref[0, 0:1, :] = jnp.zeros((1, _DM), jnp.float32)

        def stage(l, carry):
            m, cov = carry
            q = jnp.tanh(jnp.dot(m, Wq_ref[...],
                                 preferred_element_type=jnp.float32)
                         + bq_ref[...])
            s = (jnp.dot(q, keysT, preferred_element_type=jnp.float32)
                 * inv_sqrt_da + pos - cov)
            s = s - jnp.max(s, axis=1, keepdims=True)
            e = jnp.exp(s)
            y = e / jnp.sum(e, axis=1, keepdims=True)
            z = jnp.dot(y, xf, preferred_element_type=jnp.float32)
            a = jnp.tanh(jnp.dot(z, Wz_ref[...],
                                 preferred_element_type=jnp.float32))
            gi = jnp.dot(a, Wihm,
                         preferred_element_type=jnp.float32) + gi_fb
            gh = jnp.dot(m, Whh_ref[...],
                         preferred_element_type=jnp.float32) + bhh_ref[...]
            r = jax.nn.sigmoid(gi[:, :_DM] + gh[:, :_DM])
            zz = jax.nn.sigmoid(gi[:, _DM:2 * _DM] + gh[:, _DM:2 * _DM])
            n = jnp.tanh(gi[:, 2 * _DM:] + r * gh[:, 2 * _DM:])
            m2 = (1.0 - zz) * n + zz * m
            ay_ref[0, pl.ds(l, 1), :] = y
            mem_ref[0, pl.ds(l + 1, 1), :] = m2
            return m2, cov + y

        lax.fori_loop(0, _L, stage,
                      (jnp.zeros((1, _DM), jnp.float32),
                       jnp.zeros((1, T), jnp.float32)))


def _combine_body(ay_ref, F_ref, nb_ref, Wp_ref, bp_ref, tok_ref, ys_ref):
    ay = ay_ref[0]
    nbsum = nb_ref[0][:, :_KL]
    lift = jnp.tanh(F_ref[0] + nbsum * (1.0 / _K))
    mass = jnp.clip(jnp.sum(ay, axis=1, keepdims=True), 1e-6, None)
    cloud = jnp.dot(ay, lift, preferred_element_type=jnp.float32) / mass
    tok_ref[0] = jnp.dot(cloud, Wp_ref[...],
                         preferred_element_type=jnp.float32) + bp_ref[...]
    ys_ref[0] = jnp.mean(ay, axis=0, keepdims=True)


def _sc_gather_mean(table, idx):
    M = idx.shape[0] // _K
    info = plsc.get_sparse_core_info()
    NW = info.num_cores * info.num_subcores
    t_per_w = M // NW
    mesh = plsc.VectorSubcoreMesh(core_axis_name="c", subcore_axis_name="s")

    @functools.partial(
        pl.kernel, mesh=mesh,
        out_type=jax.ShapeDtypeStruct((M, _PW), jnp.float32),
        scratch_types=[
            pltpu.VMEM((_K * t_per_w,), jnp.int32),
            pltpu.VMEM((_K * t_per_w, _PW), jnp.float32),
            pltpu.VMEM((t_per_w, _PW), jnp.float32),
            pltpu.SemaphoreType.DMA,
        ],
    )
    def gather_k(table_hbm, idx_hbm, out_hbm, idx_v, rows_v, acc_v, sem):
        wid = lax.axis_index("s") * info.num_cores + lax.axis_index("c")
        pltpu.sync_copy(idx_hbm.at[pl.ds(wid * (_K * t_per_w), _K * t_per_w)],
                        idx_v)
        pltpu.async_copy(table_hbm.at[idx_v], rows_v, sem).wait()

        def tok(i, _):
            for j in range(_KL // 16):
                sl = pl.ds(j * 16, 16)
                acc_v[i, sl] = (rows_v[4 * i, sl] + rows_v[4 * i + 1, sl]
                                + rows_v[4 * i + 2, sl] + rows_v[4 * i + 3, sl])
            return 0

        lax.fori_loop(0, t_per_w, tok, 0)
        pltpu.sync_copy(acc_v, out_hbm.at[pl.ds(wid * t_per_w, t_per_w)])

    return gather_k(table, idx)


def kernel(x, feedback, W_u, b_u, W_q, b_q, W_k, b_k, pos_bias, W_z,
           W_ih, W_hh, b_ih, b_hh, lift_mu, lift_sigma, W_lift, b_lift,
           W_proj, b_proj):
    B, T, D = x.shape
    AD = W_lift.shape[0]
    DMODEL = W_proj.shape[1]

    fb3 = feedback.reshape(B, 1, 1)
    pos2 = pos_bias[:T][None, :]
    sig2, mu2 = lift_sigma[:, None], lift_mu[:, None]
    bl2 = b_lift[None, :]
    bu2, bk2, bq2 = b_u[None, :], b_k[None, :], b_q[None, :]
    bih2, bhh2, bp2 = b_ih[None, :], b_hh[None, :], b_proj[None, :]

    nblk = T // _BT
    gidx, F, P, all_y, all_memory = pl.pallas_call(
        _geom_body,
        grid=(B, nblk),
        in_specs=[
            pl.BlockSpec((1, _BT, D), lambda b, i: (b, i, 0)),
            pl.BlockSpec((1, T, D), lambda b, i: (b, 0, 0)),
            pl.BlockSpec((AD, _KL), lambda b, i: (0, 0)),
            pl.BlockSpec((AD, 1), lambda b, i: (0, 0)),
            pl.BlockSpec((AD, 1), lambda b, i: (0, 0)),
            pl.BlockSpec((1, _KL), lambda b, i: (0, 0)),
            pl.BlockSpec((1, 1, 1), lambda b, i: (b, 0, 0)),
            pl.BlockSpec((D, _DM), lambda b, i: (0, 0)),
            pl.BlockSpec((1, _DM), lambda b, i: (0, 0)),
            pl.BlockSpec((_DM, _DA), lambda b, i: (0, 0)),
            pl.BlockSpec((1, _DA), lambda b, i: (0, 0)),
            pl.BlockSpec((_DM, _DA), lambda b, i: (0, 0)),
            pl.BlockSpec((1, _DA), lambda b, i: (0, 0)),
            pl.BlockSpec((1, T), lambda b, i: (0, 0)),
            pl.BlockSpec((D, _DM), lambda b, i: (0, 0)),
            pl.BlockSpec((_DM + 1, 3 * _DM), lambda b, i: (0, 0)),
            pl.BlockSpec((1, 3 * _DM), lambda b, i: (0, 0)),
            pl.BlockSpec((_DM, 3 * _DM), lambda b, i: (0, 0)),
            pl.BlockSpec((1, 3 * _DM), lambda b, i: (0, 0)),
        ],
        out_specs=[
            pl.BlockSpec((1, _BT, _K), lambda b, i: (b, i, 0)),
            pl.BlockSpec((1, _BT, _KL), lambda b, i: (b, i, 0)),
            pl.BlockSpec((1, _BT, _PW), lambda b, i: (b, i, 0)),
            pl.BlockSpec((1, _L, T), lambda b, i: (b, 0, 0)),
            pl.BlockSpec((1, _L + 1, _DM), lambda b, i: (b, 0, 0)),
        ],
        out_shape=[
            jax.ShapeDtypeStruct((B, T, _K), jnp.int32),
            jax.ShapeDtypeStruct((B, T, _KL), jnp.float32),
            jax.ShapeDtypeStruct((B, T, _PW), jnp.float32),
            jax.ShapeDtypeStruct((B, _L, T), jnp.float32),
            jax.ShapeDtypeStruct((B, _L + 1, _DM), jnp.float32),
        ],
        scratch_shapes=[
            pltpu.VMEM((AD, _KL), jnp.float32),
            pltpu.VMEM((1, _KL), jnp.float32),
        ],
    )(x, x, W_lift, sig2, mu2, bl2, fb3, W_u, bu2, W_k, bk2, W_q, bq2,
      pos2, W_z, W_ih, bih2, W_hh, bhh2)

    nb = _sc_gather_mean(P.reshape(B * T, _PW),
                         gidx.reshape(-1)).reshape(B, T, _PW)

    tokens, ystar3 = pl.pallas_call(
        _combine_body,
        grid=(B,),
        in_specs=[
            pl.BlockSpec((1, _L, T), lambda b: (b, 0, 0)),
            pl.BlockSpec((1, T, _KL), lambda b: (b, 0, 0)),
            pl.BlockSpec((1, T, _PW), lambda b: (b, 0, 0)),
            pl.BlockSpec((_KL, DMODEL), lambda b: (0, 0)),
            pl.BlockSpec((1, DMODEL), lambda b: (0, 0)),
        ],
        out_specs=[
            pl.BlockSpec((1, _L, DMODEL), lambda b: (b, 0, 0)),
            pl.BlockSpec((1, 1, T), lambda b: (b, 0, 0)),
        ],
        out_shape=[
            jax.ShapeDtypeStruct((B, _L, DMODEL), jnp.float32),
            jax.ShapeDtypeStruct((B, 1, T), jnp.float32),
        ],
    )(all_y, F, nb, W_proj, bp2)

    return tokens, ystar3.reshape(B, T), all_y, all_memory


# speedup vs baseline: 1.0255x; 1.0255x over previous
import functools

import jax
import jax.numpy as jnp
from jax import lax
from jax.experimental import pallas as pl
from jax.experimental.pallas import tpu as pltpu
from jax.experimental.pallas import tpu_sc as plsc

_BT = 512
_K = 4
_L = 16
_DM = 128
_DA = 64
_KL = 32
_PW = 128


def _geom_body(x_ref, xf_ref, Wl_ref, sig_ref, mu_ref, blift_ref,
               gidx_ref, F_ref, P_ref, wl_s, bl_s):
    b = pl.program_id(0)
    i = pl.program_id(1)
    xr = x_ref[0]
    xf = xf_ref[0]
    T = xf.shape[0]
    D = xf.shape[1]

    @pl.when((b == 0) & (i == 0))
    def _fold():
        wl_s[...] = Wl_ref[...] / sig_ref[...]
        bl_s[...] = blift_ref[...] - lax.dot_general(
            mu_ref[...] / sig_ref[...], Wl_ref[...], (((0,), (0,)), ((), ())),
            preferred_element_type=jnp.float32)

    Wl = wl_s[...]
    b_l = bl_s[...]
    sq_r = jnp.sum(xr * xr, axis=1, keepdims=True)
    sq_c = jnp.sum(xf * xf, axis=1, keepdims=True).reshape(1, T)
    G2 = lax.dot_general(-2.0 * xr, xf, (((1,), (1,)), ((), ())),
                         preferred_element_type=jnp.float32)
    col = lax.broadcasted_iota(jnp.int32, (_BT, T), 1)
    rowg = i * _BT + lax.broadcasted_iota(jnp.int32, (_BT, T), 0)
    vals = jnp.where(col == rowg, 1e9, G2 + sq_c)
    dist_acc = jnp.zeros((_BT, 1), jnp.float32)
    for k in range(_K):
        m = jnp.min(vals, axis=1, keepdims=True)
        sel = vals == m
        idxk = jnp.min(jnp.where(sel, col, T), axis=1, keepdims=True)
        dist_acc = dist_acc + jnp.sqrt(jnp.maximum(m + sq_r, 0.0))
        gidx_ref[0, :, pl.ds(k, 1)] = idxk + b * T
        vals = jnp.where(sel, 3e9, vals)
    P_ref[0, :, :_KL] = jnp.dot(xr, Wl[D:2 * D],
                                preferred_element_type=jnp.float32)
    F_ref[0] = (jnp.dot(xr, Wl[:D], preferred_element_type=jnp.float32)
                + jnp.sqrt(sq_r) * Wl[2 * D:2 * D + 1]
                + (dist_acc * (1.0 / _K)) * Wl[2 * D + 1:2 * D + 2]
                + b_l)

def _router_combine_body(x_ref, fb_ref, Wu_ref, bu_ref, Wk_ref, bk_ref,
                         Wq_ref, bq_ref, pos_ref, Wz_ref, Wih_ref, bih_ref,
                         Whh_ref, bhh_ref, F_ref, nb_ref, Wp_ref, bp_ref,
                         ay_ref, mem_ref, tok_ref, ys_ref):
    xf = x_ref[0]
    T = xf.shape[0]
    u = jnp.tanh(jnp.dot(xf, Wu_ref[...],
                         preferred_element_type=jnp.float32) + bu_ref[...])
    keys = jnp.dot(u, Wk_ref[...],
                   preferred_element_type=jnp.float32) + bk_ref[...]
    keysT = keys.T
    fb = fb_ref[0]
    Wihm = Wih_ref[:_DM]
    gi_fb = fb * Wih_ref[_DM:_DM + 1] + bih_ref[...]
    pos = pos_ref[...]
    inv_sqrt_da = 1.0 / (_DA ** 0.5)
    mem_ref[0, 0:1, :] = jnp.zeros((1, _DM), jnp.float32)

    def stage(l, carry):
        m, cov = carry
        q = jnp.tanh(jnp.dot(m, Wq_ref[...],
                             preferred_element_type=jnp.float32) + bq_ref[...])
        s = (jnp.dot(q, keysT, preferred_element_type=jnp.float32)
             * inv_sqrt_da + pos - cov)
        s = s - jnp.max(s, axis=1, keepdims=True)
        e = jnp.exp(s)
        y = e / jnp.sum(e, axis=1, keepdims=True)
        z = jnp.dot(y, xf, preferred_element_type=jnp.float32)
        a = jnp.tanh(jnp.dot(z, Wz_ref[...],
                             preferred_element_type=jnp.float32))
        gi = jnp.dot(a, Wihm,
                     preferred_element_type=jnp.float32) + gi_fb
        gh = jnp.dot(m, Whh_ref[...],
                     preferred_element_type=jnp.float32) + bhh_ref[...]
        r = jax.nn.sigmoid(gi[:, :_DM] + gh[:, :_DM])
        zz = jax.nn.sigmoid(gi[:, _DM:2 * _DM] + gh[:, _DM:2 * _DM])
        n = jnp.tanh(gi[:, 2 * _DM:] + r * gh[:, 2 * _DM:])
        m2 = (1.0 - zz) * n + zz * m
        ay_ref[0, pl.ds(l, 1), :] = y
        mem_ref[0, pl.ds(l + 1, 1), :] = m2
        return m2, cov + y

    lax.fori_loop(0, _L, stage,
                  (jnp.zeros((1, _DM), jnp.float32),
                   jnp.zeros((1, T), jnp.float32)))

    ay = ay_ref[0]
    nbsum = nb_ref[0][:, :_KL]
    lift = jnp.tanh(F_ref[0] + nbsum * (1.0 / _K))
    mass = jnp.clip(jnp.sum(ay, axis=1, keepdims=True), 1e-6, None)
    cloud = jnp.dot(ay, lift, preferred_element_type=jnp.float32) / mass
    tok_ref[0] = jnp.dot(cloud, Wp_ref[...],
                         preferred_element_type=jnp.float32) + bp_ref[...]
    ys_ref[0] = jnp.mean(ay, axis=0, keepdims=True)


def _sc_gather_mean(table, idx):
    M = idx.shape[0] // _K
    info = plsc.get_sparse_core_info()
    NW = info.num_cores * info.num_subcores
    t_per_w = M // NW
    mesh = plsc.VectorSubcoreMesh(core_axis_name="c", subcore_axis_name="s")

    @functools.partial(
        pl.kernel, mesh=mesh,
        out_type=jax.ShapeDtypeStruct((M, _PW), jnp.float32),
        scratch_types=[
            pltpu.VMEM((_K * t_per_w,), jnp.int32),
            pltpu.VMEM((_K * t_per_w, _PW), jnp.float32),
            pltpu.VMEM((t_per_w, _PW), jnp.float32),
            pltpu.SemaphoreType.DMA,
        ],
    )
    def gather_k(table_hbm, idx_hbm, out_hbm, idx_v, rows_v, acc_v, sem):
        wid = lax.axis_index("s") * info.num_cores + lax.axis_index("c")
        pltpu.sync_copy(idx_hbm.at[pl.ds(wid * (_K * t_per_w), _K * t_per_w)],
                        idx_v)
        pltpu.async_copy(table_hbm.at[idx_v], rows_v, sem).wait()

        def tok(i, _):
            for j in range(_KL // 16):
                sl = pl.ds(j * 16, 16)
                acc_v[i, sl] = (rows_v[4 * i, sl] + rows_v[4 * i + 1, sl]
                                + rows_v[4 * i + 2, sl] + rows_v[4 * i + 3, sl])
            return 0

        lax.fori_loop(0, t_per_w, tok, 0)
        pltpu.sync_copy(acc_v, out_hbm.at[pl.ds(wid * t_per_w, t_per_w)])

    return gather_k(table, idx)


def kernel(x, feedback, W_u, b_u, W_q, b_q, W_k, b_k, pos_bias, W_z,
           W_ih, W_hh, b_ih, b_hh, lift_mu, lift_sigma, W_lift, b_lift,
           W_proj, b_proj):
    B, T, D = x.shape
    AD = W_lift.shape[0]
    DMODEL = W_proj.shape[1]

    fb3 = feedback.reshape(B, 1, 1)
    pos2 = pos_bias[:T][None, :]
    sig2, mu2 = lift_sigma[:, None], lift_mu[:, None]
    bl2 = b_lift[None, :]
    bu2, bk2, bq2 = b_u[None, :], b_k[None, :], b_q[None, :]
    bih2, bhh2, bp2 = b_ih[None, :], b_hh[None, :], b_proj[None, :]

    nblk = T // _BT
    gidx, F, P = pl.pallas_call(
        _geom_body,
        grid=(B, nblk),
        in_specs=[
            pl.BlockSpec((1, _BT, D), lambda b, i: (b, i, 0)),
            pl.BlockSpec((1, T, D), lambda b, i: (b, 0, 0)),
            pl.BlockSpec((AD, _KL), lambda b, i: (0, 0)),
            pl.BlockSpec((AD, 1), lambda b, i: (0, 0)),
            pl.BlockSpec((AD, 1), lambda b, i: (0, 0)),
            pl.BlockSpec((1, _KL), lambda b, i: (0, 0)),
        ],
        out_specs=[
            pl.BlockSpec((1, _BT, _K), lambda b, i: (b, i, 0)),
            pl.BlockSpec((1, _BT, _KL), lambda b, i: (b, i, 0)),
            pl.BlockSpec((1, _BT, _PW), lambda b, i: (b, i, 0)),
        ],
        out_shape=[
            jax.ShapeDtypeStruct((B, T, _K), jnp.int32),
            jax.ShapeDtypeStruct((B, T, _KL), jnp.float32),
            jax.ShapeDtypeStruct((B, T, _PW), jnp.float32),
        ],
        scratch_shapes=[
            pltpu.VMEM((AD, _KL), jnp.float32),
            pltpu.VMEM((1, _KL), jnp.float32),
        ],
    )(x, x, W_lift, sig2, mu2, bl2)

    nb = _sc_gather_mean(P.reshape(B * T, _PW),
                         gidx.reshape(-1)).reshape(B, T, _PW)

    all_y, all_memory, tokens, ystar3 = pl.pallas_call(
        _router_combine_body,
        grid=(B,),
        in_specs=[
            pl.BlockSpec((1, T, D), lambda b: (b, 0, 0)),
            pl.BlockSpec((1, 1, 1), lambda b: (b, 0, 0)),
            pl.BlockSpec((D, _DM), lambda b: (0, 0)),
            pl.BlockSpec((1, _DM), lambda b: (0, 0)),
            pl.BlockSpec((_DM, _DA), lambda b: (0, 0)),
            pl.BlockSpec((1, _DA), lambda b: (0, 0)),
            pl.BlockSpec((_DM, _DA), lambda b: (0, 0)),
            pl.BlockSpec((1, _DA), lambda b: (0, 0)),
            pl.BlockSpec((1, T), lambda b: (0, 0)),
            pl.BlockSpec((D, _DM), lambda b: (0, 0)),
            pl.BlockSpec((_DM + 1, 3 * _DM), lambda b: (0, 0)),
            pl.BlockSpec((1, 3 * _DM), lambda b: (0, 0)),
            pl.BlockSpec((_DM, 3 * _DM), lambda b: (0, 0)),
            pl.BlockSpec((1, 3 * _DM), lambda b: (0, 0)),
            pl.BlockSpec((1, T, _KL), lambda b: (b, 0, 0)),
            pl.BlockSpec((1, T, _PW), lambda b: (b, 0, 0)),
            pl.BlockSpec((_KL, DMODEL), lambda b: (0, 0)),
            pl.BlockSpec((1, DMODEL), lambda b: (0, 0)),
        ],
        out_specs=[
            pl.BlockSpec((1, _L, T), lambda b: (b, 0, 0)),
            pl.BlockSpec((1, _L + 1, _DM), lambda b: (b, 0, 0)),
            pl.BlockSpec((1, _L, DMODEL), lambda b: (b, 0, 0)),
            pl.BlockSpec((1, 1, T), lambda b: (b, 0, 0)),
        ],
        out_shape=[
            jax.ShapeDtypeStruct((B, _L, T), jnp.float32),
            jax.ShapeDtypeStruct((B, _L + 1, _DM), jnp.float32),
            jax.ShapeDtypeStruct((B, _L, DMODEL), jnp.float32),
            jax.ShapeDtypeStruct((B, 1, T), jnp.float32),
        ],
    )(x, fb3, W_u, bu2, W_k, bk2, W_q, bq2, pos2, W_z, W_ih, bih2,
      W_hh, bhh2, F, nb, W_proj, bp2)

    return tokens, ystar3.reshape(B, T), all_y, all_memory


# speedup vs baseline: 1.1610x; 1.1321x over previous
import functools

import jax
import jax.numpy as jnp
from jax import lax
from jax.experimental import pallas as pl
from jax.experimental.pallas import tpu as pltpu
from jax.experimental.pallas import tpu_sc as plsc

_BT = 512
_K = 4
_L = 16
_DM = 128
_DA = 64
_KL = 32
_PW = 128


def _geom_body(x_ref, xf_ref, Wl_ref, sig_ref, mu_ref, blift_ref,
               gidx_ref, F_ref, P_ref, wl_s, bl_s):
    b = pl.program_id(0)
    i = pl.program_id(1)
    xr = x_ref[0]
    xf = xf_ref[0]
    T = xf.shape[0]
    D = xf.shape[1]

    @pl.when((b == 0) & (i == 0))
    def _fold():
        wl_s[...] = Wl_ref[...] / sig_ref[...]
        bl_s[...] = blift_ref[...] - lax.dot_general(
            mu_ref[...] / sig_ref[...], Wl_ref[...], (((0,), (0,)), ((), ())),
            preferred_element_type=jnp.float32)

    Wl = wl_s[...]
    b_l = bl_s[...]
    sq_r = jnp.sum(xr * xr, axis=1, keepdims=True)
    sq_c = jnp.sum(xf * xf, axis=1, keepdims=True).reshape(1, T)
    G2 = lax.dot_general(-2.0 * xr, xf, (((1,), (1,)), ((), ())),
                         preferred_element_type=jnp.float32)
    col = lax.broadcasted_iota(jnp.int32, (_BT, T), 1)
    rowg = i * _BT + lax.broadcasted_iota(jnp.int32, (_BT, T), 0)
    vals = jnp.where(col == rowg, 1e9, G2 + sq_c)
    dist_acc = jnp.zeros((_BT, 1), jnp.float32)
    for k in range(_K):
        m = jnp.min(vals, axis=1, keepdims=True)
        sel = vals == m
        idxk = jnp.min(jnp.where(sel, col, T), axis=1, keepdims=True)
        dist_acc = dist_acc + jnp.sqrt(jnp.maximum(m + sq_r, 0.0))
        gidx_ref[0, :, pl.ds(k, 1)] = idxk + b * T
        vals = jnp.where(sel, 3e9, vals)
    P_ref[0, :, :_KL] = jnp.dot(xr, Wl[D:2 * D],
                                preferred_element_type=jnp.float32)
    F_ref[0] = (jnp.dot(xr, Wl[:D], preferred_element_type=jnp.float32)
                + jnp.sqrt(sq_r) * Wl[2 * D:2 * D + 1]
                + (dist_acc * (1.0 / _K)) * Wl[2 * D + 1:2 * D + 2]
                + b_l)

def _router_body(x_ref, fb_ref, Wu_ref, bu_ref, Wk_ref, bk_ref, Wq_ref,
                 bq_ref, pos_ref, Wz_ref, Wih_ref, bih_ref,
                 Whh_ref, bhh_ref, ay_ref, mem_ref):
    x = x_ref[...]
    BT2 = x.shape[0]
    T = BT2 // 2
    u = jnp.tanh(jnp.dot(x, Wu_ref[...],
                         preferred_element_type=jnp.float32) + bu_ref[...])
    keys = jnp.dot(u, Wk_ref[...],
                   preferred_element_type=jnp.float32) + bk_ref[...]
    k0T = keys[:T].T
    k1T = keys[T:].T
    fb = fb_ref[...].reshape(2, 1)
    Wihm = Wih_ref[:_DM]
    gi_fb = fb * Wih_ref[_DM:_DM + 1] + bih_ref[...]
    pos = pos_ref[...]
    inv_sqrt_da = 1.0 / (_DA ** 0.5)
    mem_ref[:, 0:1, :] = jnp.zeros((2, 1, _DM), jnp.float32)

    def stage(l, carry):
        m, cov = carry
        q = jnp.tanh(jnp.dot(m, Wq_ref[...],
                             preferred_element_type=jnp.float32) + bq_ref[...])
        s0 = jnp.dot(q[0:1], k0T, preferred_element_type=jnp.float32)
        s1 = jnp.dot(q[1:2], k1T, preferred_element_type=jnp.float32)
        s = (jnp.concatenate([s0, s1], axis=0) * inv_sqrt_da + pos - cov)
        s = s - jnp.max(s, axis=1, keepdims=True)
        e = jnp.exp(s)
        y = e / jnp.sum(e, axis=1, keepdims=True)
        z0 = jnp.dot(y[0:1], x[:T], preferred_element_type=jnp.float32)
        z1 = jnp.dot(y[1:2], x[T:], preferred_element_type=jnp.float32)
        z = jnp.concatenate([z0, z1], axis=0)
        a = jnp.tanh(jnp.dot(z, Wz_ref[...],
                             preferred_element_type=jnp.float32))
        gi = jnp.dot(a, Wihm,
                     preferred_element_type=jnp.float32) + gi_fb
        gh = jnp.dot(m, Whh_ref[...],
                     preferred_element_type=jnp.float32) + bhh_ref[...]
        r = jax.nn.sigmoid(gi[:, :_DM] + gh[:, :_DM])
        zz = jax.nn.sigmoid(gi[:, _DM:2 * _DM] + gh[:, _DM:2 * _DM])
        n = jnp.tanh(gi[:, 2 * _DM:] + r * gh[:, 2 * _DM:])
        m2 = (1.0 - zz) * n + zz * m
        ay_ref[:, pl.ds(l, 1), :] = y.reshape(2, 1, T)
        mem_ref[:, pl.ds(l + 1, 1), :] = m2.reshape(2, 1, _DM)
        return m2, cov + y

    lax.fori_loop(0, _L, stage,
                  (jnp.zeros((2, _DM), jnp.float32),
                   jnp.zeros((2, T), jnp.float32)))


def _combine_body(ay_ref, F_ref, nb_ref, Wp_ref, bp_ref, tok_ref, ys_ref):
    ay = ay_ref[0]
    nbsum = nb_ref[0][:, :_KL]
    lift = jnp.tanh(F_ref[0] + nbsum * (1.0 / _K))
    mass = jnp.clip(jnp.sum(ay, axis=1, keepdims=True), 1e-6, None)
    cloud = jnp.dot(ay, lift, preferred_element_type=jnp.float32) / mass
    tok_ref[0] = jnp.dot(cloud, Wp_ref[...],
                         preferred_element_type=jnp.float32) + bp_ref[...]
    ys_ref[0] = jnp.mean(ay, axis=0, keepdims=True)


def _sc_gather_mean(table, idx):
    M = idx.shape[0] // _K
    info = plsc.get_sparse_core_info()
    NW = info.num_cores * info.num_subcores
    t_per_w = M // NW
    mesh = plsc.VectorSubcoreMesh(core_axis_name="c", subcore_axis_name="s")

    @functools.partial(
        pl.kernel, mesh=mesh,
        out_type=jax.ShapeDtypeStruct((M, _PW), jnp.float32),
        scratch_types=[
            pltpu.VMEM((_K * t_per_w,), jnp.int32),
            pltpu.VMEM((_K * t_per_w, _PW), jnp.float32),
            pltpu.VMEM((t_per_w, _PW), jnp.float32),
            pltpu.SemaphoreType.DMA,
        ],
    )
    def gather_k(table_hbm, idx_hbm, out_hbm, idx_v, rows_v, acc_v, sem):
        wid = lax.axis_index("s") * info.num_cores + lax.axis_index("c")
        pltpu.sync_copy(idx_hbm.at[pl.ds(wid * (_K * t_per_w), _K * t_per_w)],
                        idx_v)
        pltpu.async_copy(table_hbm.at[idx_v], rows_v, sem).wait()

        def tok(i, _):
            for j in range(_KL // 16):
                sl = pl.ds(j * 16, 16)
                acc_v[i, sl] = (rows_v[4 * i, sl] + rows_v[4 * i + 1, sl]
                                + rows_v[4 * i + 2, sl] + rows_v[4 * i + 3, sl])
            return 0

        lax.fori_loop(0, t_per_w, tok, 0)
        pltpu.sync_copy(acc_v, out_hbm.at[pl.ds(wid * t_per_w, t_per_w)])

    return gather_k(table, idx)


def kernel(x, feedback, W_u, b_u, W_q, b_q, W_k, b_k, pos_bias, W_z,
           W_ih, W_hh, b_ih, b_hh, lift_mu, lift_sigma, W_lift, b_lift,
           W_proj, b_proj):
    B, T, D = x.shape
    AD = W_lift.shape[0]
    DMODEL = W_proj.shape[1]

    fb3 = feedback.reshape(B, 1, 1)
    pos2 = pos_bias[:T][None, :]
    sig2, mu2 = lift_sigma[:, None], lift_mu[:, None]
    bl2 = b_lift[None, :]
    bu2, bk2, bq2 = b_u[None, :], b_k[None, :], b_q[None, :]
    bih2, bhh2, bp2 = b_ih[None, :], b_hh[None, :], b_proj[None, :]

    nblk = T // _BT
    gidx, F, P = pl.pallas_call(
        _geom_body,
        grid=(B, nblk),
        in_specs=[
            pl.BlockSpec((1, _BT, D), lambda b, i: (b, i, 0)),
            pl.BlockSpec((1, T, D), lambda b, i: (b, 0, 0)),
            pl.BlockSpec((AD, _KL), lambda b, i: (0, 0)),
            pl.BlockSpec((AD, 1), lambda b, i: (0, 0)),
            pl.BlockSpec((AD, 1), lambda b, i: (0, 0)),
            pl.BlockSpec((1, _KL), lambda b, i: (0, 0)),
        ],
        out_specs=[
            pl.BlockSpec((1, _BT, _K), lambda b, i: (b, i, 0)),
            pl.BlockSpec((1, _BT, _KL), lambda b, i: (b, i, 0)),
            pl.BlockSpec((1, _BT, _PW), lambda b, i: (b, i, 0)),
        ],
        out_shape=[
            jax.ShapeDtypeStruct((B, T, _K), jnp.int32),
            jax.ShapeDtypeStruct((B, T, _KL), jnp.float32),
            jax.ShapeDtypeStruct((B, T, _PW), jnp.float32),
        ],
        scratch_shapes=[
            pltpu.VMEM((AD, _KL), jnp.float32),
            pltpu.VMEM((1, _KL), jnp.float32),
        ],
    )(x, x, W_lift, sig2, mu2, bl2)

    nb = _sc_gather_mean(P.reshape(B * T, _PW),
                         gidx.reshape(-1)).reshape(B, T, _PW)

    x2d = x.reshape(B * T, D)
    all_y, all_memory = pl.pallas_call(
        _router_body,
        grid=(1,),
        in_specs=[
            pl.BlockSpec((B * T, D), lambda g: (0, 0)),
            pl.BlockSpec((B, 1, 1), lambda g: (0, 0, 0)),
            pl.BlockSpec((D, _DM), lambda g: (0, 0)),
            pl.BlockSpec((1, _DM), lambda g: (0, 0)),
            pl.BlockSpec((_DM, _DA), lambda g: (0, 0)),
            pl.BlockSpec((1, _DA), lambda g: (0, 0)),
            pl.BlockSpec((_DM, _DA), lambda g: (0, 0)),
            pl.BlockSpec((1, _DA), lambda g: (0, 0)),
            pl.BlockSpec((1, T), lambda g: (0, 0)),
            pl.BlockSpec((D, _DM), lambda g: (0, 0)),
            pl.BlockSpec((_DM + 1, 3 * _DM), lambda g: (0, 0)),
            pl.BlockSpec((1, 3 * _DM), lambda g: (0, 0)),
            pl.BlockSpec((_DM, 3 * _DM), lambda g: (0, 0)),
            pl.BlockSpec((1, 3 * _DM), lambda g: (0, 0)),
        ],
        out_specs=[
            pl.BlockSpec((B, _L, T), lambda g: (0, 0, 0)),
            pl.BlockSpec((B, _L + 1, _DM), lambda g: (0, 0, 0)),
        ],
        out_shape=[
            jax.ShapeDtypeStruct((B, _L, T), jnp.float32),
            jax.ShapeDtypeStruct((B, _L + 1, _DM), jnp.float32),
        ],
    )(x2d, fb3, W_u, bu2, W_k, bk2, W_q, bq2, pos2, W_z, W_ih, bih2,
      W_hh, bhh2)

    tokens, ystar3 = pl.pallas_call(
        _combine_body,
        grid=(B,),
        in_specs=[
            pl.BlockSpec((1, _L, T), lambda b: (b, 0, 0)),
            pl.BlockSpec((1, T, _KL), lambda b: (b, 0, 0)),
            pl.BlockSpec((1, T, _PW), lambda b: (b, 0, 0)),
            pl.BlockSpec((_KL, DMODEL), lambda b: (0, 0)),
            pl.BlockSpec((1, DMODEL), lambda b: (0, 0)),
        ],
        out_specs=[
            pl.BlockSpec((1, _L, DMODEL), lambda b: (b, 0, 0)),
            pl.BlockSpec((1, 1, T), lambda b: (b, 0, 0)),
        ],
        out_shape=[
            jax.ShapeDtypeStruct((B, _L, DMODEL), jnp.float32),
            jax.ShapeDtypeStruct((B, 1, T), jnp.float32),
        ],
    )(all_y, F, nb, W_proj, bp2)

    return tokens, ystar3.reshape(B, T), all_y, all_memory


# speedup vs baseline: 1.1935x; 1.0280x over previous
import functools

import jax
import jax.numpy as jnp
from jax import lax
from jax.experimental import pallas as pl
from jax.experimental.pallas import tpu as pltpu
from jax.experimental.pallas import tpu_sc as plsc

_BT = 1024
_K = 4
_L = 16
_DM = 128
_DA = 64
_KL = 32
_PW = 128


def _geom_body(x_ref, xf_ref, Wl_ref, sig_ref, mu_ref, blift_ref,
               gidx_ref, F_ref, P_ref, wl_s, bl_s):
    b = pl.program_id(0)
    i = pl.program_id(1)
    xr = x_ref[0]
    xf = xf_ref[0]
    T = xf.shape[0]
    D = xf.shape[1]

    @pl.when((b == 0) & (i == 0))
    def _fold():
        wl_s[...] = Wl_ref[...] / sig_ref[...]
        bl_s[...] = blift_ref[...] - lax.dot_general(
            mu_ref[...] / sig_ref[...], Wl_ref[...], (((0,), (0,)), ((), ())),
            preferred_element_type=jnp.float32)

    Wl = wl_s[...]
    b_l = bl_s[...]
    sq_r = jnp.sum(xr * xr, axis=1, keepdims=True)
    sq_c = jnp.sum(xf * xf, axis=1, keepdims=True).reshape(1, T)
    G2 = lax.dot_general(-2.0 * xr, xf, (((1,), (1,)), ((), ())),
                         preferred_element_type=jnp.float32)
    col = lax.broadcasted_iota(jnp.int32, (_BT, T), 1)
    rowg = i * _BT + lax.broadcasted_iota(jnp.int32, (_BT, T), 0)
    vals = jnp.where(col == rowg, 1e9, G2 + sq_c)
    dist_acc = jnp.zeros((_BT, 1), jnp.float32)
    for k in range(_K):
        m = jnp.min(vals, axis=1, keepdims=True)
        sel = vals == m
        idxk = jnp.min(jnp.where(sel, col, T), axis=1, keepdims=True)
        dist_acc = dist_acc + jnp.sqrt(jnp.maximum(m + sq_r, 0.0))
        gidx_ref[0, :, pl.ds(k, 1)] = idxk + b * T
        vals = jnp.where(sel, 3e9, vals)
    P_ref[0, :, :_KL] = jnp.dot(xr, Wl[D:2 * D],
                                preferred_element_type=jnp.float32)
    F_ref[0] = (jnp.dot(xr, Wl[:D], preferred_element_type=jnp.float32)
                + jnp.sqrt(sq_r) * Wl[2 * D:2 * D + 1]
                + (dist_acc * (1.0 / _K)) * Wl[2 * D + 1:2 * D + 2]
                + b_l)

def _router_body(x_ref, fb_ref, Wu_ref, bu_ref, Wk_ref, bk_ref, Wq_ref,
                 bq_ref, pos_ref, Wz_ref, Wih_ref, bih_ref,
                 Whh_ref, bhh_ref, ay_ref, mem_ref):
    x = x_ref[...]
    BT2 = x.shape[0]
    T = BT2 // 2
    u = jnp.tanh(jnp.dot(x, Wu_ref[...],
                         preferred_element_type=jnp.float32) + bu_ref[...])
    keys = jnp.dot(u, Wk_ref[...],
                   preferred_element_type=jnp.float32) + bk_ref[...]
    k0T = keys[:T].T
    k1T = keys[T:].T
    fb = fb_ref[...].reshape(2, 1)
    Wihm = Wih_ref[:_DM]
    gi_fb = fb * Wih_ref[_DM:_DM + 1] + bih_ref[...]
    pos = pos_ref[...]
    inv_sqrt_da = 1.0 / (_DA ** 0.5)
    mem_ref[:, 0:1, :] = jnp.zeros((2, 1, _DM), jnp.float32)

    def stage(l, carry):
        m, cov = carry
        q = jnp.tanh(jnp.dot(m, Wq_ref[...],
                             preferred_element_type=jnp.float32) + bq_ref[...])
        s0 = jnp.dot(q[0:1], k0T, preferred_element_type=jnp.float32)
        s1 = jnp.dot(q[1:2], k1T, preferred_element_type=jnp.float32)
        s = (jnp.concatenate([s0, s1], axis=0) * inv_sqrt_da + pos - cov)
        s = s - jnp.max(s, axis=1, keepdims=True)
        e = jnp.exp(s)
        y = e / jnp.sum(e, axis=1, keepdims=True)
        z0 = jnp.dot(y[0:1], x[:T], preferred_element_type=jnp.float32)
        z1 = jnp.dot(y[1:2], x[T:], preferred_element_type=jnp.float32)
        z = jnp.concatenate([z0, z1], axis=0)
        a = jnp.tanh(jnp.dot(z, Wz_ref[...],
                             preferred_element_type=jnp.float32))
        gi = jnp.dot(a, Wihm,
                     preferred_element_type=jnp.float32) + gi_fb
        gh = jnp.dot(m, Whh_ref[...],
                     preferred_element_type=jnp.float32) + bhh_ref[...]
        r = jax.nn.sigmoid(gi[:, :_DM] + gh[:, :_DM])
        zz = jax.nn.sigmoid(gi[:, _DM:2 * _DM] + gh[:, _DM:2 * _DM])
        n = jnp.tanh(gi[:, 2 * _DM:] + r * gh[:, 2 * _DM:])
        m2 = (1.0 - zz) * n + zz * m
        ay_ref[:, pl.ds(l, 1), :] = y.reshape(2, 1, T)
        mem_ref[:, pl.ds(l + 1, 1), :] = m2.reshape(2, 1, _DM)
        return m2, cov + y

    lax.fori_loop(0, _L, stage,
                  (jnp.zeros((2, _DM), jnp.float32),
                   jnp.zeros((2, T), jnp.float32)))


def _combine_body(ay_ref, F_ref, nb_ref, Wp_ref, bp_ref, tok_ref, ys_ref):
    ay = ay_ref[0]
    nbsum = nb_ref[0][:, :_KL]
    lift = jnp.tanh(F_ref[0] + nbsum * (1.0 / _K))
    mass = jnp.clip(jnp.sum(ay, axis=1, keepdims=True), 1e-6, None)
    cloud = jnp.dot(ay, lift, preferred_element_type=jnp.float32) / mass
    tok_ref[0] = jnp.dot(cloud, Wp_ref[...],
                         preferred_element_type=jnp.float32) + bp_ref[...]
    ys_ref[0] = jnp.mean(ay, axis=0, keepdims=True)


def _sc_gather_mean(table, idx):
    M = idx.shape[0] // _K
    info = plsc.get_sparse_core_info()
    NW = info.num_cores * info.num_subcores
    t_per_w = M // NW
    mesh = plsc.VectorSubcoreMesh(core_axis_name="c", subcore_axis_name="s")

    @functools.partial(
        pl.kernel, mesh=mesh,
        out_type=jax.ShapeDtypeStruct((M, _PW), jnp.float32),
        scratch_types=[
            pltpu.VMEM((_K * t_per_w,), jnp.int32),
            pltpu.VMEM((_K * t_per_w, _PW), jnp.float32),
            pltpu.VMEM((t_per_w, _PW), jnp.float32),
            pltpu.SemaphoreType.DMA,
        ],
    )
    def gather_k(table_hbm, idx_hbm, out_hbm, idx_v, rows_v, acc_v, sem):
        wid = lax.axis_index("s") * info.num_cores + lax.axis_index("c")
        pltpu.sync_copy(idx_hbm.at[pl.ds(wid * (_K * t_per_w), _K * t_per_w)],
                        idx_v)
        pltpu.async_copy(table_hbm.at[idx_v], rows_v, sem).wait()

        def tok(i, _):
            for j in range(_KL // 16):
                sl = pl.ds(j * 16, 16)
                acc_v[i, sl] = (rows_v[4 * i, sl] + rows_v[4 * i + 1, sl]
                                + rows_v[4 * i + 2, sl] + rows_v[4 * i + 3, sl])
            return 0

        lax.fori_loop(0, t_per_w, tok, 0)
        pltpu.sync_copy(acc_v, out_hbm.at[pl.ds(wid * t_per_w, t_per_w)])

    return gather_k(table, idx)


def kernel(x, feedback, W_u, b_u, W_q, b_q, W_k, b_k, pos_bias, W_z,
           W_ih, W_hh, b_ih, b_hh, lift_mu, lift_sigma, W_lift, b_lift,
           W_proj, b_proj):
    B, T, D = x.shape
    AD = W_lift.shape[0]
    DMODEL = W_proj.shape[1]

    fb3 = feedback.reshape(B, 1, 1)
    pos2 = pos_bias[:T][None, :]
    sig2, mu2 = lift_sigma[:, None], lift_mu[:, None]
    bl2 = b_lift[None, :]
    bu2, bk2, bq2 = b_u[None, :], b_k[None, :], b_q[None, :]
    bih2, bhh2, bp2 = b_ih[None, :], b_hh[None, :], b_proj[None, :]

    nblk = T // _BT
    gidx, F, P = pl.pallas_call(
        _geom_body,
        grid=(B, nblk),
        in_specs=[
            pl.BlockSpec((1, _BT, D), lambda b, i: (b, i, 0)),
            pl.BlockSpec((1, T, D), lambda b, i: (b, 0, 0)),
            pl.BlockSpec((AD, _KL), lambda b, i: (0, 0)),
            pl.BlockSpec((AD, 1), lambda b, i: (0, 0)),
            pl.BlockSpec((AD, 1), lambda b, i: (0, 0)),
            pl.BlockSpec((1, _KL), lambda b, i: (0, 0)),
        ],
        out_specs=[
            pl.BlockSpec((1, _BT, _K), lambda b, i: (b, i, 0)),
            pl.BlockSpec((1, _BT, _KL), lambda b, i: (b, i, 0)),
            pl.BlockSpec((1, _BT, _PW), lambda b, i: (b, i, 0)),
        ],
        out_shape=[
            jax.ShapeDtypeStruct((B, T, _K), jnp.int32),
            jax.ShapeDtypeStruct((B, T, _KL), jnp.float32),
            jax.ShapeDtypeStruct((B, T, _PW), jnp.float32),
        ],
        scratch_shapes=[
            pltpu.VMEM((AD, _KL), jnp.float32),
            pltpu.VMEM((1, _KL), jnp.float32),
        ],
    )(x, x, W_lift, sig2, mu2, bl2)

    nb = _sc_gather_mean(P.reshape(B * T, _PW),
                         gidx.reshape(-1)).reshape(B, T, _PW)

    x2d = x.reshape(B * T, D)
    all_y, all_memory = pl.pallas_call(
        _router_body,
        grid=(1,),
        in_specs=[
            pl.BlockSpec((B * T, D), lambda g: (0, 0)),
            pl.BlockSpec((B, 1, 1), lambda g: (0, 0, 0)),
            pl.BlockSpec((D, _DM), lambda g: (0, 0)),
            pl.BlockSpec((1, _DM), lambda g: (0, 0)),
            pl.BlockSpec((_DM, _DA), lambda g: (0, 0)),
            pl.BlockSpec((1, _DA), lambda g: (0, 0)),
            pl.BlockSpec((_DM, _DA), lambda g: (0, 0)),
            pl.BlockSpec((1, _DA), lambda g: (0, 0)),
            pl.BlockSpec((1, T), lambda g: (0, 0)),
            pl.BlockSpec((D, _DM), lambda g: (0, 0)),
            pl.BlockSpec((_DM + 1, 3 * _DM), lambda g: (0, 0)),
            pl.BlockSpec((1, 3 * _DM), lambda g: (0, 0)),
            pl.BlockSpec((_DM, 3 * _DM), lambda g: (0, 0)),
            pl.BlockSpec((1, 3 * _DM), lambda g: (0, 0)),
        ],
        out_specs=[
            pl.BlockSpec((B, _L, T), lambda g: (0, 0, 0)),
            pl.BlockSpec((B, _L + 1, _DM), lambda g: (0, 0, 0)),
        ],
        out_shape=[
            jax.ShapeDtypeStruct((B, _L, T), jnp.float32),
            jax.ShapeDtypeStruct((B, _L + 1, _DM), jnp.float32),
        ],
    )(x2d, fb3, W_u, bu2, W_k, bk2, W_q, bq2, pos2, W_z, W_ih, bih2,
      W_hh, bhh2)

    tokens, ystar3 = pl.pallas_call(
        _combine_body,
        grid=(B,),
        in_specs=[
            pl.BlockSpec((1, _L, T), lambda b: (b, 0, 0)),
            pl.BlockSpec((1, T, _KL), lambda b: (b, 0, 0)),
            pl.BlockSpec((1, T, _PW), lambda b: (b, 0, 0)),
            pl.BlockSpec((_KL, DMODEL), lambda b: (0, 0)),
            pl.BlockSpec((1, DMODEL), lambda b: (0, 0)),
        ],
        out_specs=[
            pl.BlockSpec((1, _L, DMODEL), lambda b: (b, 0, 0)),
            pl.BlockSpec((1, 1, T), lambda b: (b, 0, 0)),
        ],
        out_shape=[
            jax.ShapeDtypeStruct((B, _L, DMODEL), jnp.float32),
            jax.ShapeDtypeStruct((B, 1, T), jnp.float32),
        ],
    )(all_y, F, nb, W_proj, bp2)

    return tokens, ystar3.reshape(B, T), all_y, all_memory
